# baseline (speedup 1.0000x reference)
import functools

import jax
import jax.numpy as jnp
from jax.experimental import pallas as pl
from jax.experimental.pallas import tpu as pltpu

NU = 50000
NI = 50000
ND = 64
N = NU + NI


def _precompute_body(x_ref, wd_ref, wb_ref, b_ref, a_ref, bb_ref):
    x = x_ref[...]
    a_ref[...] = x @ wd_ref[...] + b_ref[...]
    bb_ref[...] = x @ wb_ref[...]


def _node_precompute(x, wd, wb, b, bn):
    n, d = x.shape
    dout = wd.shape[1]
    grid = n // bn
    return pl.pallas_call(
        _precompute_body,
        grid=(grid,),
        in_specs=[
            pl.BlockSpec((bn, d), lambda i: (i, 0)),
            pl.BlockSpec((d, dout), lambda i: (0, 0)),
            pl.BlockSpec((d, dout), lambda i: (0, 0)),
            pl.BlockSpec((1, dout), lambda i: (0, 0)),
        ],
        out_specs=[
            pl.BlockSpec((bn, dout), lambda i: (i, 0)),
            pl.BlockSpec((bn, dout), lambda i: (i, 0)),
        ],
        out_shape=[
            jax.ShapeDtypeStruct((n, dout), jnp.float32),
            jax.ShapeDtypeStruct((n, dout), jnp.float32),
        ],
    )(x, wd, wb, b)


def _mlp2_body(g_ref, w_ref, b_ref, o_ref):
    o_ref[...] = jnp.maximum(g_ref[...], 0.0) @ w_ref[...] + b_ref[...]


def _edge_mlp2(g, w, b, be):
    e, d = g.shape
    dout = w.shape[1]
    return pl.pallas_call(
        _mlp2_body,
        grid=(e // be,),
        in_specs=[
            pl.BlockSpec((be, d), lambda i: (i, 0)),
            pl.BlockSpec((d, dout), lambda i: (0, 0)),
            pl.BlockSpec((1, dout), lambda i: (0, 0)),
        ],
        out_specs=pl.BlockSpec((be, dout), lambda i: (i, 0)),
        out_shape=jax.ShapeDtypeStruct((e, dout), jnp.float32),
    )(g, w, b)


def _loss_body(u_ref, p_ref, n_ref, o_ref):
    u = u_ref[...]
    diff = jnp.sum(u * (p_ref[...] - n_ref[...]), axis=1, keepdims=True)
    o_ref[...] = -jax.nn.log_sigmoid(diff)


def _bpr_loss(u, p, n):
    b, d = u.shape
    out = pl.pallas_call(
        _loss_body,
        grid=(1,),
        in_specs=[pl.BlockSpec((b, d), lambda i: (0, 0))] * 3,
        out_specs=pl.BlockSpec((b, 1), lambda i: (0, 0)),
        out_shape=jax.ShapeDtypeStruct((b, 1), jnp.float32),
    )(u, p, n)
    return jnp.mean(out)


def kernel(X, user_emb, item_emb, W1, b1, W2, b2, W3, b3, W4, b4, edge_index):
    feats = jnp.concatenate([user_emb, item_emb], axis=0)
    src = edge_index[0]
    dst = edge_index[1]

    w1a, w1b = W1[:ND], W1[ND:]
    A1, B1 = _node_precompute(feats, w1a - w1b, w1b, b1[None, :], 5000)
    g1 = A1[dst] + B1[src]
    m1 = _edge_mlp2(g1, W2, b2[None, :], 4000)
    h = jax.ops.segment_max(m1, dst, num_segments=N)
    h = jnp.maximum(jnp.where(jnp.isfinite(h), h, 0.0), 0.0)

    w3a, w3b = W3[: 2 * ND], W3[2 * ND :]
    A2, B2 = _node_precompute(h, w3a - w3b, w3b, b3[None, :], 5000)
    g2 = A2[dst] + B2[src]
    m2 = _edge_mlp2(g2, W4, b4[None, :], 4000)
    out = jax.ops.segment_max(m2, dst, num_segments=N)
    out = jnp.where(jnp.isfinite(out), out, 0.0)

    uids, piids, niids = X[:, 0], X[:, 1], X[:, 2]
    u = out[uids]
    p = out[NU + piids]
    n = out[NU + niids]
    return _bpr_loss(u, p, n)


# speedup vs baseline: 1.0986x; 1.0986x over previous
import functools

import jax
import jax.numpy as jnp
from jax import lax
from jax.experimental import pallas as pl
from jax.experimental.pallas import tpu as pltpu
from jax.experimental.pallas import tpu_sc as plsc

NU = 50000
NI = 50000
ND = 64
N = NU + NI
E = 500000
NW = 32
K = 128
E2 = 516096


def _gather2_kernel(ta, ia, tb, ib, oa, ob, iav, ibv, bufa, bufb, sga, sgb, swa, swb):
    info = plsc.get_sparse_core_info()
    nc = info.num_cores
    wid = lax.axis_index("s") * nc + lax.axis_index("c")
    per_w = iav.shape[0]
    nchunks = per_w // K
    base = wid * per_w
    pltpu.sync_copy(ia.at[pl.ds(base, per_w)], iav)
    pltpu.sync_copy(ib.at[pl.ds(base, per_w)], ibv)

    def body(j, carry):
        descs = []
        for s in range(2):
            off = (2 * j + s) * K
            ga = pltpu.async_copy(ta.at[iav.at[pl.ds(off, K)]], bufa.at[s], sga)
            gb = pltpu.async_copy(tb.at[ibv.at[pl.ds(off, K)]], bufb.at[s], sgb)
            descs.append((ga, gb))
        writes = []
        for s in range(2):
            off = (2 * j + s) * K
            ga, gb = descs[s]
            ga.wait()
            wa = pltpu.async_copy(bufa.at[s], oa.at[pl.ds(base + off, K)], swa)
            gb.wait()
            wb = pltpu.async_copy(bufb.at[s], ob.at[pl.ds(base + off, K)], swb)
            writes.append((wa, wb))
        for wa, wb in writes:
            wa.wait()
            wb.wait()
        return carry

    lax.fori_loop(0, nchunks // 2, body, 0)


def _gather2(ta, ia, tb, ib):
    n_idx = ia.shape[0]
    d = ta.shape[1]
    per_w = n_idx // NW
    mesh = plsc.VectorSubcoreMesh(core_axis_name="c", subcore_axis_name="s")
    f = functools.partial(
        pl.kernel,
        mesh=mesh,
        out_type=[
            jax.ShapeDtypeStruct((n_idx, d), jnp.float32),
            jax.ShapeDtypeStruct((n_idx, d), jnp.float32),
        ],
        scratch_types=[
            pltpu.VMEM((per_w,), jnp.int32),
            pltpu.VMEM((per_w,), jnp.int32),
            pltpu.VMEM((2, K, d), jnp.float32),
            pltpu.VMEM((2, K, d), jnp.float32),
            pltpu.SemaphoreType.DMA,
            pltpu.SemaphoreType.DMA,
            pltpu.SemaphoreType.DMA,
            pltpu.SemaphoreType.DMA,
        ],
    )(_gather2_kernel)
    return f(ta, ia, tb, ib)


def _precompute_body(x_ref, wd_ref, wb_ref, b_ref, a_ref, bb_ref):
    x = x_ref[...]
    a_ref[...] = x @ wd_ref[...] + b_ref[...]
    bb_ref[...] = x @ wb_ref[...]


def _node_precompute(x, wd, wb, b, bn):
    n, d = x.shape
    dout = wd.shape[1]
    return pl.pallas_call(
        _precompute_body,
        grid=(n // bn,),
        in_specs=[
            pl.BlockSpec((bn, d), lambda i: (i, 0)),
            pl.BlockSpec((d, dout), lambda i: (0, 0)),
            pl.BlockSpec((d, dout), lambda i: (0, 0)),
            pl.BlockSpec((1, dout), lambda i: (0, 0)),
        ],
        out_specs=[
            pl.BlockSpec((bn, dout), lambda i: (i, 0)),
            pl.BlockSpec((bn, dout), lambda i: (i, 0)),
        ],
        out_shape=[
            jax.ShapeDtypeStruct((n, dout), jnp.float32),
            jax.ShapeDtypeStruct((n, dout), jnp.float32),
        ],
    )(x, wd, wb, b)


def _mlp1_body(ga_ref, gb_ref, w_ref, b_ref, o_ref):
    pre = ga_ref[...] + gb_ref[...]
    o_ref[...] = jnp.maximum(pre, 0.0) @ w_ref[...] + b_ref[...]


def _edge_mlp1(ga, gb, w, b, be):
    e, d = ga.shape
    dout = w.shape[1]
    return pl.pallas_call(
        _mlp1_body,
        grid=(e // be,),
        in_specs=[
            pl.BlockSpec((be, d), lambda i: (i, 0)),
            pl.BlockSpec((be, d), lambda i: (i, 0)),
            pl.BlockSpec((d, dout), lambda i: (0, 0)),
            pl.BlockSpec((1, dout), lambda i: (0, 0)),
        ],
        out_specs=pl.BlockSpec((be, dout), lambda i: (i, 0)),
        out_shape=jax.ShapeDtypeStruct((e, dout), jnp.float32),
    )(ga, gb, w, b)


def _mlp2_body(gi_ref, gj_ref, wd_ref, wb_ref, b3_ref, w4_ref, b4_ref, o_ref):
    x_i = jnp.maximum(gi_ref[...], 0.0)
    x_j = jnp.maximum(gj_ref[...], 0.0)
    pre = x_i @ wd_ref[...] + x_j @ wb_ref[...] + b3_ref[...]
    o_ref[...] = jnp.maximum(pre, 0.0) @ w4_ref[...] + b4_ref[...]


def _edge_mlp2(gi, gj, wd, wb, b3, w4, b4, be):
    e, d = gi.shape
    dout = wd.shape[1]
    return pl.pallas_call(
        _mlp2_body,
        grid=(e // be,),
        in_specs=[
            pl.BlockSpec((be, d), lambda i: (i, 0)),
            pl.BlockSpec((be, d), lambda i: (i, 0)),
            pl.BlockSpec((d, dout), lambda i: (0, 0)),
            pl.BlockSpec((d, dout), lambda i: (0, 0)),
            pl.BlockSpec((1, dout), lambda i: (0, 0)),
            pl.BlockSpec((dout, dout), lambda i: (0, 0)),
            pl.BlockSpec((1, dout), lambda i: (0, 0)),
        ],
        out_specs=pl.BlockSpec((be, dout), lambda i: (i, 0)),
        out_shape=jax.ShapeDtypeStruct((e, dout), jnp.float32),
    )(gi, gj, wd, wb, b3, w4, b4)


def _loss_body(u_ref, p_ref, n_ref, o_ref):
    u = u_ref[...]
    diff = jnp.sum(u * (p_ref[...] - n_ref[...]), axis=1, keepdims=True)
    o_ref[...] = -jax.nn.log_sigmoid(diff)


def _bpr_loss(u, p, n):
    b, d = u.shape
    out = pl.pallas_call(
        _loss_body,
        grid=(1,),
        in_specs=[pl.BlockSpec((b, d), lambda i: (0, 0))] * 3,
        out_specs=pl.BlockSpec((b, 1), lambda i: (0, 0)),
        out_shape=jax.ShapeDtypeStruct((b, 1), jnp.float32),
    )(u, p, n)
    return jnp.mean(out)


def kernel(X, user_emb, item_emb, W1, b1, W2, b2, W3, b3, W4, b4, edge_index):
    feats = jnp.concatenate([user_emb, item_emb], axis=0)
    src = edge_index[0].astype(jnp.int32)
    dst = edge_index[1].astype(jnp.int32)
    dstp = jnp.pad(dst, (0, E2 - E))
    srcp = jnp.pad(src, (0, E2 - E))

    w1d = W1[:ND] - W1[ND:]
    w1b = W1[ND:]
    A1, B1 = _node_precompute(feats, w1d, w1b, b1[None, :], 5000)
    ga1, gb1 = _gather2(A1, dstp, B1, srcp)
    m1 = _edge_mlp1(ga1, gb1, W2, b2[None, :], 4032)
    h = jax.ops.segment_max(m1[:E], dst, num_segments=N)

    w3d = W3[: 2 * ND] - W3[2 * ND :]
    w3b = W3[2 * ND :]
    gh_i, gh_j = _gather2(h, dstp, h, srcp)
    m2 = _edge_mlp2(gh_i, gh_j, w3d, w3b, b3[None, :], W4, b4[None, :], 4032)
    out = jax.ops.segment_max(m2[:E], dst, num_segments=N)
    out = jnp.where(jnp.isfinite(out), out, 0.0)

    uids = X[:, 0].astype(jnp.int32)
    piids = X[:, 1].astype(jnp.int32) + NU
    niids = X[:, 2].astype(jnp.int32) + NU
    B = uids.shape[0]
    outp = jnp.pad(out, ((0, 0), (0, 2 * ND - ND)))
    ia = jnp.concatenate([uids, piids])
    ib = jnp.concatenate([niids, jnp.zeros((B,), jnp.int32)])
    up, nrows = _gather2(outp, ia, outp, ib)
    return _bpr_loss(up[:B, :ND], up[B:, :ND], nrows[:B, :ND])


# speedup vs baseline: 1.7675x; 1.6090x over previous
import functools

import jax
import jax.numpy as jnp
from jax import lax
from jax.experimental import pallas as pl
from jax.experimental.pallas import tpu as pltpu
from jax.experimental.pallas import tpu_sc as plsc

NU = 50000
NI = 50000
ND = 64
N = NU + NI
E = 500000
NW = 32
K = 128
E2 = 516096


def _gather2_kernel(ta, ia, tb, ib, oa, ob, iav, ibv, bufa, bufb, sga, sgb, swa, swb):
    info = plsc.get_sparse_core_info()
    nc = info.num_cores
    wid = lax.axis_index("s") * nc + lax.axis_index("c")
    per_w = iav.shape[0]
    nchunks = per_w // K
    base = wid * per_w
    pltpu.sync_copy(ia.at[pl.ds(base, per_w)], iav)
    pltpu.sync_copy(ib.at[pl.ds(base, per_w)], ibv)

    def body(j, carry):
        descs = []
        for s in range(2):
            off = (2 * j + s) * K
            ga = pltpu.async_copy(ta.at[iav.at[pl.ds(off, K)]], bufa.at[s], sga)
            gb = pltpu.async_copy(tb.at[ibv.at[pl.ds(off, K)]], bufb.at[s], sgb)
            descs.append((ga, gb))
        writes = []
        for s in range(2):
            off = (2 * j + s) * K
            ga, gb = descs[s]
            ga.wait()
            wa = pltpu.async_copy(bufa.at[s], oa.at[pl.ds(base + off, K)], swa)
            gb.wait()
            wb = pltpu.async_copy(bufb.at[s], ob.at[pl.ds(base + off, K)], swb)
            writes.append((wa, wb))
        for wa, wb in writes:
            wa.wait()
            wb.wait()
        return carry

    lax.fori_loop(0, nchunks // 2, body, 0)


def _gather2(ta, ia, tb, ib):
    n_idx = ia.shape[0]
    d = ta.shape[1]
    per_w = n_idx // NW
    mesh = plsc.VectorSubcoreMesh(core_axis_name="c", subcore_axis_name="s")
    f = functools.partial(
        pl.kernel,
        mesh=mesh,
        out_type=[
            jax.ShapeDtypeStruct((n_idx, d), jnp.float32),
            jax.ShapeDtypeStruct((n_idx, d), jnp.float32),
        ],
        scratch_types=[
            pltpu.VMEM((per_w,), jnp.int32),
            pltpu.VMEM((per_w,), jnp.int32),
            pltpu.VMEM((2, K, d), jnp.float32),
            pltpu.VMEM((2, K, d), jnp.float32),
            pltpu.SemaphoreType.DMA,
            pltpu.SemaphoreType.DMA,
            pltpu.SemaphoreType.DMA,
            pltpu.SemaphoreType.DMA,
        ],
    )(_gather2_kernel)
    return f(ta, ia, tb, ib)


def _precompute_body(x_ref, wd_ref, wb_ref, b_ref, a_ref, bb_ref):
    x = x_ref[...]
    a_ref[...] = x @ wd_ref[...] + b_ref[...]
    bb_ref[...] = x @ wb_ref[...]


def _node_precompute(x, wd, wb, b, bn):
    n, d = x.shape
    dout = wd.shape[1]
    return pl.pallas_call(
        _precompute_body,
        grid=(n // bn,),
        in_specs=[
            pl.BlockSpec((bn, d), lambda i: (i, 0)),
            pl.BlockSpec((d, dout), lambda i: (0, 0)),
            pl.BlockSpec((d, dout), lambda i: (0, 0)),
            pl.BlockSpec((1, dout), lambda i: (0, 0)),
        ],
        out_specs=[
            pl.BlockSpec((bn, dout), lambda i: (i, 0)),
            pl.BlockSpec((bn, dout), lambda i: (i, 0)),
        ],
        out_shape=[
            jax.ShapeDtypeStruct((n, dout), jnp.float32),
            jax.ShapeDtypeStruct((n, dout), jnp.float32),
        ],
    )(x, wd, wb, b)


def _mlp1_body(ga_ref, gb_ref, w_ref, b_ref, o_ref):
    pre = ga_ref[...] + gb_ref[...]
    o_ref[...] = jnp.maximum(pre, 0.0) @ w_ref[...] + b_ref[...]


def _edge_mlp1(ga, gb, w, b, be):
    e, d = ga.shape
    dout = w.shape[1]
    return pl.pallas_call(
        _mlp1_body,
        grid=(e // be,),
        in_specs=[
            pl.BlockSpec((be, d), lambda i: (i, 0)),
            pl.BlockSpec((be, d), lambda i: (i, 0)),
            pl.BlockSpec((d, dout), lambda i: (0, 0)),
            pl.BlockSpec((1, dout), lambda i: (0, 0)),
        ],
        out_specs=pl.BlockSpec((be, dout), lambda i: (i, 0)),
        out_shape=jax.ShapeDtypeStruct((e, dout), jnp.float32),
    )(ga, gb, w, b)


def _mlp2_body(gi_ref, gj_ref, wd_ref, wb_ref, b3_ref, w4_ref, b4_ref, o_ref):
    x_i = jnp.maximum(gi_ref[...], 0.0)
    x_j = jnp.maximum(gj_ref[...], 0.0)
    pre = x_i @ wd_ref[...] + x_j @ wb_ref[...] + b3_ref[...]
    o_ref[...] = jnp.maximum(pre, 0.0) @ w4_ref[...] + b4_ref[...]


def _edge_mlp2(gi, gj, wd, wb, b3, w4, b4, be):
    e, d = gi.shape
    dout = wd.shape[1]
    return pl.pallas_call(
        _mlp2_body,
        grid=(e // be,),
        in_specs=[
            pl.BlockSpec((be, d), lambda i: (i, 0)),
            pl.BlockSpec((be, d), lambda i: (i, 0)),
            pl.BlockSpec((d, dout), lambda i: (0, 0)),
            pl.BlockSpec((d, dout), lambda i: (0, 0)),
            pl.BlockSpec((1, dout), lambda i: (0, 0)),
            pl.BlockSpec((dout, dout), lambda i: (0, 0)),
            pl.BlockSpec((1, dout), lambda i: (0, 0)),
        ],
        out_specs=pl.BlockSpec((be, dout), lambda i: (i, 0)),
        out_shape=jax.ShapeDtypeStruct((e, dout), jnp.float32),
    )(gi, gj, wd, wb, b3, w4, b4)


def _loss_body(u_ref, p_ref, n_ref, o_ref):
    u = u_ref[...]
    diff = jnp.sum(u * (p_ref[...] - n_ref[...]), axis=1, keepdims=True)
    o_ref[...] = -jax.nn.log_sigmoid(diff)


def _bpr_loss(u, p, n):
    b, d = u.shape
    out = pl.pallas_call(
        _loss_body,
        grid=(1,),
        in_specs=[pl.BlockSpec((b, d), lambda i: (0, 0))] * 3,
        out_specs=pl.BlockSpec((b, 1), lambda i: (0, 0)),
        out_shape=jax.ShapeDtypeStruct((b, 1), jnp.float32),
    )(u, p, n)
    return jnp.mean(out)


def kernel(X, user_emb, item_emb, W1, b1, W2, b2, W3, b3, W4, b4, edge_index):
    feats = jnp.concatenate([user_emb, item_emb], axis=0)
    src = edge_index[0].astype(jnp.int32)
    dst = edge_index[1].astype(jnp.int32)
    spread = (jnp.arange(E2 - E, dtype=jnp.int32) * 41) % N
    dstp = jnp.concatenate([dst, spread])
    srcp = jnp.concatenate([src, spread])

    w1d = W1[:ND] - W1[ND:]
    w1b = W1[ND:]
    A1, B1 = _node_precompute(feats, w1d, w1b, b1[None, :], 5000)
    ga1, gb1 = _gather2(A1, dstp, B1, srcp)
    m1 = _edge_mlp1(ga1, gb1, W2, b2[None, :], 4032)
    h = jax.ops.segment_max(m1[:E], dst, num_segments=N)

    w3d = W3[: 2 * ND] - W3[2 * ND :]
    w3b = W3[2 * ND :]
    gh_i, gh_j = _gather2(h, dstp, h, srcp)
    m2 = _edge_mlp2(gh_i, gh_j, w3d, w3b, b3[None, :], W4, b4[None, :], 4032)
    out = jax.ops.segment_max(m2[:E], dst, num_segments=N)
    out = jnp.where(jnp.isfinite(out), out, 0.0)

    uids = X[:, 0].astype(jnp.int32)
    piids = X[:, 1].astype(jnp.int32) + NU
    niids = X[:, 2].astype(jnp.int32) + NU
    B = uids.shape[0]
    outp = jnp.pad(out, ((0, 0), (0, 2 * ND - ND)))
    ia = jnp.concatenate([uids, piids])
    ib = jnp.concatenate([niids, (jnp.arange(B, dtype=jnp.int32) * 23) % N])
    up, nrows = _gather2(outp, ia, outp, ib)
    return _bpr_loss(up[:B, :ND], up[B:, :ND], nrows[:B, :ND])


# speedup vs baseline: 1.9311x; 1.0926x over previous
import functools

import jax
import jax.numpy as jnp
from jax import lax
from jax.experimental import pallas as pl
from jax.experimental.pallas import tpu as pltpu
from jax.experimental.pallas import tpu_sc as plsc

NU = 50000
NI = 50000
ND = 64
N = NU + NI
E = 500000
NW = 32
K = 128
E2 = 516096


def _gather2_kernel(ta, ia, tb, ib, oa, ob, iav, ibv, bufa, bufb, sga, sgb, swa, swb):
    info = plsc.get_sparse_core_info()
    nc = info.num_cores
    wid = lax.axis_index("s") * nc + lax.axis_index("c")
    per_w = iav.shape[0]
    nchunks = per_w // K
    base = wid * per_w
    pltpu.sync_copy(ia.at[pl.ds(base, per_w)], iav)
    pltpu.sync_copy(ib.at[pl.ds(base, per_w)], ibv)

    def body(j, carry):
        descs = []
        for s in range(2):
            off = (2 * j + s) * K
            ga = pltpu.async_copy(ta.at[iav.at[pl.ds(off, K)]], bufa.at[s], sga)
            gb = pltpu.async_copy(tb.at[ibv.at[pl.ds(off, K)]], bufb.at[s], sgb)
            descs.append((ga, gb))
        writes = []
        for s in range(2):
            off = (2 * j + s) * K
            ga, gb = descs[s]
            ga.wait()
            wa = pltpu.async_copy(bufa.at[s], oa.at[pl.ds(base + off, K)], swa)
            gb.wait()
            wb = pltpu.async_copy(bufb.at[s], ob.at[pl.ds(base + off, K)], swb)
            writes.append((wa, wb))
        for wa, wb in writes:
            wa.wait()
            wb.wait()
        return carry

    lax.fori_loop(0, nchunks // 2, body, 0)


def _gather2(ta, ia, tb, ib):
    n_idx = ia.shape[0]
    d = ta.shape[1]
    per_w = n_idx // NW
    mesh = plsc.VectorSubcoreMesh(core_axis_name="c", subcore_axis_name="s")
    f = functools.partial(
        pl.kernel,
        mesh=mesh,
        out_type=[
            jax.ShapeDtypeStruct((n_idx, d), jnp.float32),
            jax.ShapeDtypeStruct((n_idx, d), jnp.float32),
        ],
        scratch_types=[
            pltpu.VMEM((per_w,), jnp.int32),
            pltpu.VMEM((per_w,), jnp.int32),
            pltpu.VMEM((2, K, d), jnp.float32),
            pltpu.VMEM((2, K, d), jnp.float32),
            pltpu.SemaphoreType.DMA,
            pltpu.SemaphoreType.DMA,
            pltpu.SemaphoreType.DMA,
            pltpu.SemaphoreType.DMA,
        ],
    )(_gather2_kernel)
    return f(ta, ia, tb, ib)


def _precompute_body(x_ref, wd_ref, wb_ref, b_ref, a_ref, bb_ref):
    x = x_ref[...]
    a_ref[...] = x @ wd_ref[...] + b_ref[...]
    bb_ref[...] = x @ wb_ref[...]


def _node_precompute(x, wd, wb, b, bn):
    n, d = x.shape
    dout = wd.shape[1]
    return pl.pallas_call(
        _precompute_body,
        grid=(n // bn,),
        in_specs=[
            pl.BlockSpec((bn, d), lambda i: (i, 0)),
            pl.BlockSpec((d, dout), lambda i: (0, 0)),
            pl.BlockSpec((d, dout), lambda i: (0, 0)),
            pl.BlockSpec((1, dout), lambda i: (0, 0)),
        ],
        out_specs=[
            pl.BlockSpec((bn, dout), lambda i: (i, 0)),
            pl.BlockSpec((bn, dout), lambda i: (i, 0)),
        ],
        out_shape=[
            jax.ShapeDtypeStruct((n, dout), jnp.float32),
            jax.ShapeDtypeStruct((n, dout), jnp.float32),
        ],
    )(x, wd, wb, b)


def _mlp1_body(ga_ref, gb_ref, w_ref, b_ref, o_ref):
    pre = ga_ref[...] + gb_ref[...]
    o_ref[...] = jnp.maximum(pre, 0.0) @ w_ref[...] + b_ref[...]


def _edge_mlp1(ga, gb, w, b, be, e_out):
    d = ga.shape[1]
    dout = w.shape[1]
    return pl.pallas_call(
        _mlp1_body,
        grid=(e_out // be,),
        in_specs=[
            pl.BlockSpec((be, d), lambda i: (i, 0)),
            pl.BlockSpec((be, d), lambda i: (i, 0)),
            pl.BlockSpec((d, dout), lambda i: (0, 0)),
            pl.BlockSpec((1, dout), lambda i: (0, 0)),
        ],
        out_specs=pl.BlockSpec((be, dout), lambda i: (i, 0)),
        out_shape=jax.ShapeDtypeStruct((e_out, dout), jnp.float32),
    )(ga, gb, w, b)


def _mlp2_body(gi_ref, gj_ref, wd_ref, wb_ref, b3_ref, w4_ref, b4_ref, o_ref):
    x_i = jnp.maximum(gi_ref[...], 0.0)
    x_j = jnp.maximum(gj_ref[...], 0.0)
    pre = x_i @ wd_ref[...] + x_j @ wb_ref[...] + b3_ref[...]
    o_ref[...] = jnp.maximum(pre, 0.0) @ w4_ref[...] + b4_ref[...]


def _edge_mlp2(gi, gj, wd, wb, b3, w4, b4, be, e_out):
    d = gi.shape[1]
    dout = wd.shape[1]
    return pl.pallas_call(
        _mlp2_body,
        grid=(e_out // be,),
        in_specs=[
            pl.BlockSpec((be, d), lambda i: (i, 0)),
            pl.BlockSpec((be, d), lambda i: (i, 0)),
            pl.BlockSpec((d, dout), lambda i: (0, 0)),
            pl.BlockSpec((d, dout), lambda i: (0, 0)),
            pl.BlockSpec((1, dout), lambda i: (0, 0)),
            pl.BlockSpec((dout, dout), lambda i: (0, 0)),
            pl.BlockSpec((1, dout), lambda i: (0, 0)),
        ],
        out_specs=pl.BlockSpec((be, dout), lambda i: (i, 0)),
        out_shape=jax.ShapeDtypeStruct((e_out, dout), jnp.float32),
    )(gi, gj, wd, wb, b3, w4, b4)


def _loss_body(u_ref, p_ref, n_ref, o_ref):
    u = jnp.where(u_ref[...] > -1e38, u_ref[...], 0.0)
    p = jnp.where(p_ref[...] > -1e38, p_ref[...], 0.0)
    n = jnp.where(n_ref[...] > -1e38, n_ref[...], 0.0)
    diff = jnp.sum(u * (p - n), axis=1, keepdims=True)
    o_ref[...] = -jax.nn.log_sigmoid(diff)


def _bpr_loss(u, p, n):
    b, d = u.shape
    out = pl.pallas_call(
        _loss_body,
        grid=(1,),
        in_specs=[pl.BlockSpec((b, d), lambda i: (0, 0))] * 3,
        out_specs=pl.BlockSpec((b, 1), lambda i: (0, 0)),
        out_shape=jax.ShapeDtypeStruct((b, 1), jnp.float32),
    )(u, p, n)
    return jnp.mean(out)


def kernel(X, user_emb, item_emb, W1, b1, W2, b2, W3, b3, W4, b4, edge_index):
    feats = jnp.concatenate([user_emb, item_emb], axis=0)
    src = edge_index[0].astype(jnp.int32)
    dst = edge_index[1].astype(jnp.int32)
    spread = (jnp.arange(E2 - E, dtype=jnp.int32) * 41) % N
    dstp = jnp.concatenate([dst, spread])
    srcp = jnp.concatenate([src, spread])

    w1d = W1[:ND] - W1[ND:]
    w1b = W1[ND:]
    A1, B1 = _node_precompute(feats, w1d, w1b, b1[None, :], 5000)
    ga1, gb1 = _gather2(A1, dstp, B1, srcp)
    m1 = _edge_mlp1(ga1, gb1, W2, b2[None, :], 4000, E)
    h = jax.ops.segment_max(m1, dst, num_segments=N)

    w3d = W3[: 2 * ND] - W3[2 * ND :]
    w3b = W3[2 * ND :]
    gh_i, gh_j = _gather2(h, dstp, h, srcp)
    m2 = _edge_mlp2(gh_i, gh_j, w3d, w3b, b3[None, :], W4, b4[None, :], 4000, E)
    out = jax.ops.segment_max(m2, dst, num_segments=N)

    uids = X[:, 0].astype(jnp.int32)
    piids = X[:, 1].astype(jnp.int32) + NU
    niids = X[:, 2].astype(jnp.int32) + NU
    B = uids.shape[0]
    outp = jnp.pad(out, ((0, 0), (0, 2 * ND - ND)))
    ia = jnp.concatenate([uids, piids])
    ib = jnp.concatenate([niids, (jnp.arange(B, dtype=jnp.int32) * 23) % N])
    up, nrows = _gather2(outp, ia, outp, ib)
    return _bpr_loss(up[:B, :ND], up[B:, :ND], nrows[:B, :ND])
